# baseline (speedup 1.0000x reference)
import functools

import jax
import jax.numpy as jnp
from jax import lax
from jax.experimental import pallas as pl
from jax.experimental.pallas import tpu as pltpu

D_MODEL = 4096
D_SAE = 32768
N_TOK = 4096
TOPK = 100

BM = 1024
BN = 1024
BK = 512


def _mm_body(x_ref, w_ref, b_ref, o_ref):
    k = pl.program_id(2)
    nk = pl.num_programs(2)
    acc = lax.dot_general(
        x_ref[...], w_ref[...],
        (((1,), (1,)), ((), ())),
        preferred_element_type=jnp.float32,
        precision=lax.Precision.DEFAULT,
    )

    @pl.when(k == 0)
    def _init():
        o_ref[...] = acc

    @pl.when(k > 0)
    def _acc():
        o_ref[...] += acc

    @pl.when(k == nk - 1)
    def _fin():
        o_ref[...] = jnp.maximum(o_ref[...] + b_ref[...], 0.0)


@functools.partial(jax.jit, static_argnames=())
def _encode_post(xc, W_enc, b2):
    grid = (N_TOK // BM, D_SAE // BN, D_MODEL // BK)
    return pl.pallas_call(
        _mm_body,
        grid=grid,
        in_specs=[
            pl.BlockSpec((BM, BK), lambda i, j, k: (i, k)),
            pl.BlockSpec((BN, BK), lambda i, j, k: (j, k)),
            pl.BlockSpec((1, BN), lambda i, j, k: (0, j)),
        ],
        out_specs=pl.BlockSpec((BM, BN), lambda i, j, k: (i, j)),
        out_shape=jax.ShapeDtypeStruct((N_TOK, D_SAE), jnp.float32),
        compiler_params=pltpu.CompilerParams(
            dimension_semantics=("parallel", "arbitrary", "arbitrary"),
        ),
    )(xc, W_enc, b2)


def kernel(x, W_enc, b_enc, dec_bias):
    xc = x - dec_bias[None, :]
    post = _encode_post(xc, W_enc, b_enc.reshape(1, -1))
    top_acts, top_indices = lax.top_k(post, TOPK)
    rows = jnp.arange(post.shape[0])[:, None]
    encoded_acts = jnp.zeros_like(post).at[rows, top_indices].set(top_acts)
    return encoded_acts, top_acts, top_indices


# speedup vs baseline: 4.2971x; 4.2971x over previous
import functools

import jax
import jax.numpy as jnp
from jax import lax
from jax.experimental import pallas as pl
from jax.experimental.pallas import tpu as pltpu
from jax.experimental.pallas import tpu_sc as plsc

D_MODEL = 4096
D_SAE = 32768
N_TOK = 4096
TOPK = 100


BM = 1024
BN = 1024
BK = 512


def _mm_body(x_ref, w_ref, b_ref, o_ref):
    k = pl.program_id(2)
    nk = pl.num_programs(2)
    acc = lax.dot_general(
        x_ref[...], w_ref[...],
        (((1,), (1,)), ((), ())),
        preferred_element_type=jnp.float32,
        precision=lax.Precision.DEFAULT,
    )

    @pl.when(k == 0)
    def _init():
        o_ref[...] = acc

    @pl.when(k > 0)
    def _acc():
        o_ref[...] += acc

    @pl.when(k == nk - 1)
    def _fin():
        o_ref[...] = jnp.maximum(o_ref[...] + b_ref[...], 0.0)


def _encode_post(xc, W_enc, b2):
    grid = (N_TOK // BM, D_SAE // BN, D_MODEL // BK)
    return pl.pallas_call(
        _mm_body,
        grid=grid,
        in_specs=[
            pl.BlockSpec((BM, BK), lambda i, j, k: (i, k)),
            pl.BlockSpec((BN, BK), lambda i, j, k: (j, k)),
            pl.BlockSpec((1, BN), lambda i, j, k: (0, j)),
        ],
        out_specs=pl.BlockSpec((BM, BN), lambda i, j, k: (i, j)),
        out_shape=jax.ShapeDtypeStruct((N_TOK, D_SAE), jnp.float32),
        compiler_params=pltpu.CompilerParams(
            dimension_semantics=("parallel", "arbitrary", "arbitrary"),
        ),
    )(xc, W_enc, b2)


NW = 32
ROWS_PER_W = N_TOK // NW
NV = D_SAE // 16
NBINS = 2048
LCAP = 64
SELW = 128
_SENT_V = -1.0
_SENT_VB = -1082130432
_SENT_I = 2**31 - 1


def _topk_select(post):
    mesh = plsc.VectorSubcoreMesh(core_axis_name="c", subcore_axis_name="s")

    @functools.partial(
        pl.kernel,
        mesh=mesh,
        compiler_params=pltpu.CompilerParams(needs_layout_passes=False),
        out_type=[
            jax.ShapeDtypeStruct((N_TOK, SELW), jnp.int32),
            jax.ShapeDtypeStruct((N_TOK, SELW), jnp.int32),
        ],
        scratch_types=[
            pltpu.VMEM((2 * D_SAE,), jnp.int32),
            pltpu.VMEM((NBINS * 16,), jnp.int32),
            pltpu.VMEM((LCAP * 16,), jnp.int32),
            pltpu.VMEM((LCAP * 16,), jnp.int32),
            pltpu.VMEM((SELW + 16,), jnp.int32),
            pltpu.VMEM((SELW + 16,), jnp.int32),
            pltpu.SemaphoreType.DMA,
            pltpu.SemaphoreType.DMA,
        ],
    )
    def sel_kernel(post_hbm, selv_hbm, seli_hbm,
                   rowbuf, hist, lanev, lanei, selv, seli, sem_in, sem_out):
        cid = lax.axis_index("c")
        sid = lax.axis_index("s")
        wid = sid * 2 + cid
        row0 = wid * ROWS_PER_W
        lanes = lax.iota(jnp.int32, 16)
        ones = jnp.ones((16,), jnp.int32)
        zero16 = jnp.zeros((16,), jnp.int32)

        def scan_down(bstart, cstart):
            def cond(st):
                b, c = st
                return c + jnp.sum(hist[pl.ds(b * 16, 16)]) < TOPK

            def body(st):
                b, c = st
                return b - 1, c + jnp.sum(hist[pl.ds(b * 16, 16)])

            return lax.while_loop(cond, body, (bstart, cstart))

        def process_row(r, base):
            @pl.when(r > row0)
            def _drain():
                pltpu.make_async_copy(
                    selv.at[pl.ds(0, SELW)], selv_hbm.at[r], sem_out).wait()
                pltpu.make_async_copy(
                    seli.at[pl.ds(0, SELW)], seli_hbm.at[r], sem_out).wait()

            def clr(j, _):
                hist[pl.ds(j * 16, 16)] = zero16
                return 0
            lax.fori_loop(0, NBINS, clr, 0, unroll=8)

            def p1(j, mx):
                u = rowbuf[pl.ds(base + j * 16, 16)]
                key = lax.shift_right_logical(u, 20)
                plsc.addupdate_scatter(hist, [key * 16 + lanes], ones)
                return jnp.maximum(mx, u)
            mx = lax.fori_loop(0, NV, p1, zero16, unroll=4)
            b1_start = lax.shift_right_logical(jnp.max(mx), 20)
            b1, c1 = scan_down(b1_start, jnp.int32(0))

            lax.fori_loop(0, NBINS, clr, 0, unroll=8)
            t1 = lax.shift_left(b1, 20)

            def p2(j, st):
                lanecnt, mx2 = st
                u = rowbuf[pl.ds(base + j * 16, 16)]
                key = lax.shift_right_logical(u, 20)
                m_bin = key == b1
                k2 = jnp.bitwise_and(lax.shift_right_logical(u, 9), 0x7FF)
                plsc.addupdate_scatter(hist, [k2 * 16 + lanes], ones, mask=m_bin)
                mx2 = jnp.maximum(mx2, jnp.where(m_bin, k2, 0))
                m_app = jnp.logical_and(u >= t1, lanecnt < LCAP)
                plsc.store_scatter(lanev, [lanecnt * 16 + lanes], u, mask=m_app)
                plsc.store_scatter(lanei, [lanecnt * 16 + lanes], j * 16 + lanes,
                                   mask=m_app)
                return lanecnt + m_app.astype(jnp.int32), mx2

            lanecnt, mx2 = lax.fori_loop(0, NV, p2, (zero16, zero16),
                                         unroll=4)
            b2, _c2 = scan_down(jnp.max(mx2), c1)
            t_lo = jnp.bitwise_or(t1, lax.shift_left(b2, 9))

            def sent(j, _):
                selv[pl.ds(j * 16, 16)] = jnp.full((16,), _SENT_VB, jnp.int32)
                seli[pl.ds(j * 16, 16)] = jnp.full((16,), _SENT_I, jnp.int32)
                return 0
            lax.fori_loop(0, (SELW + 16) // 16, sent, 0, unroll=4)

            def comp(j, cur):
                lv = lanev[pl.ds(j * 16, 16)]
                li = lanei[pl.ds(j * 16, 16)]
                m = jnp.logical_and(j < lanecnt, lv >= t_lo)
                m = jnp.logical_and(m, cur < SELW)
                plsc.store_compressed(selv.at[pl.ds(cur, 16)], lv, mask=m)
                plsc.store_compressed(seli.at[pl.ds(cur, 16)], li, mask=m)
                pc = plsc.all_reduce_population_count(m)
                return cur + jnp.max(pc)
            lax.fori_loop(0, LCAP, comp, jnp.int32(0))

            pltpu.make_async_copy(
                selv.at[pl.ds(0, SELW)], selv_hbm.at[r], sem_out).start()
            pltpu.make_async_copy(
                seli.at[pl.ds(0, SELW)], seli_hbm.at[r], sem_out).start()

        pltpu.make_async_copy(post_hbm.at[row0],
                              rowbuf.at[pl.ds(0, D_SAE)], sem_in).start()

        def outer(t, _):
            for b in (0, 1):
                i = 2 * t + b
                r = row0 + i
                pltpu.make_async_copy(
                    post_hbm.at[r], rowbuf.at[pl.ds(b * D_SAE, D_SAE)],
                    sem_in).wait()

                @pl.when(i + 1 < ROWS_PER_W)
                def _start(r=r, b=b):
                    pltpu.make_async_copy(
                        post_hbm.at[r + 1],
                        rowbuf.at[pl.ds((1 - b) * D_SAE, D_SAE)],
                        sem_in).start()

                process_row(r, b * D_SAE)
            return 0

        lax.fori_loop(0, ROWS_PER_W // 2, outer, 0)
        rlast = row0 + ROWS_PER_W - 1
        pltpu.make_async_copy(
            selv.at[pl.ds(0, SELW)], selv_hbm.at[rlast], sem_out).wait()
        pltpu.make_async_copy(
            seli.at[pl.ds(0, SELW)], seli_hbm.at[rlast], sem_out).wait()

    return sel_kernel(post)


BR = 512
BC = 2048


def _rank_body(selv_ref, seli_ref, post_ref, ta_ref, ti_ref, enc_ref, t_ref):
    j = pl.program_id(1)

    @pl.when(j == 0)
    def _sort():
        v = selv_ref[...]
        ix = seli_ref[...]
        iota = lax.broadcasted_iota(jnp.int32, (BR, SELW), 1)
        for s in range(7):
            asc = (iota & (2 << s)) == 0
            for d in [1 << t for t in range(s, -1, -1)]:
                upper = (iota & d) != 0
                pv = jnp.where(upper, pltpu.roll(v, d, 1),
                               pltpu.roll(v, SELW - d, 1))
                pi = jnp.where(upper, pltpu.roll(ix, d, 1),
                               pltpu.roll(ix, SELW - d, 1))
                self_first = (v > pv) | ((v == pv) & (ix < pi))
                take_max = upper == asc
                takes_self = jnp.logical_xor(self_first, take_max)
                v = jnp.where(takes_self, v, pv)
                ix = jnp.where(takes_self, ix, pi)
        ta_ref[...] = v
        ti_ref[...] = ix
        t_ref[...] = jnp.broadcast_to(v[:, (TOPK - 1):TOPK], (BR, SELW))

    enc_ref[...] = jnp.where(post_ref[...] >= t_ref[:, 0:1], post_ref[...],
                             0.0)


def _rank_and_mask(selv, seli, post):
    grid = (N_TOK // BR, D_SAE // BC)
    return pl.pallas_call(
        _rank_body,
        grid=grid,
        in_specs=[
            pl.BlockSpec((BR, SELW), lambda i, j: (i, 0)),
            pl.BlockSpec((BR, SELW), lambda i, j: (i, 0)),
            pl.BlockSpec((BR, BC), lambda i, j: (i, j)),
        ],
        out_specs=[
            pl.BlockSpec((BR, SELW), lambda i, j: (i, 0)),
            pl.BlockSpec((BR, SELW), lambda i, j: (i, 0)),
            pl.BlockSpec((BR, BC), lambda i, j: (i, j)),
        ],
        out_shape=[
            jax.ShapeDtypeStruct((N_TOK, SELW), jnp.float32),
            jax.ShapeDtypeStruct((N_TOK, SELW), jnp.int32),
            jax.ShapeDtypeStruct((N_TOK, D_SAE), jnp.float32),
        ],
        scratch_shapes=[pltpu.VMEM((BR, SELW), jnp.float32)],
        compiler_params=pltpu.CompilerParams(
            dimension_semantics=("parallel", "arbitrary"),
        ),
    )(selv, seli, post)


def kernel(x, W_enc, b_enc, dec_bias):
    xc = x - dec_bias[None, :]
    post = _encode_post(xc, W_enc, b_enc.reshape(1, -1))
    post_bits = lax.bitcast_convert_type(post, jnp.int32)
    selv_b, seli = _topk_select(post_bits)
    selv = lax.bitcast_convert_type(selv_b, jnp.float32)
    top_v, top_i, encoded_acts = _rank_and_mask(selv, seli, post)
    return encoded_acts, top_v[:, :TOPK], top_i[:, :TOPK]


# speedup vs baseline: 4.3183x; 1.0049x over previous
import functools

import jax
import jax.numpy as jnp
from jax import lax
from jax.experimental import pallas as pl
from jax.experimental.pallas import tpu as pltpu
from jax.experimental.pallas import tpu_sc as plsc

D_MODEL = 4096
D_SAE = 32768
N_TOK = 4096
TOPK = 100


BM = 1024
BN = 1024
BK = 512


def _mm_body(x_ref, w_ref, b_ref, o_ref):
    k = pl.program_id(2)
    nk = pl.num_programs(2)
    acc = lax.dot_general(
        x_ref[...], w_ref[...],
        (((1,), (1,)), ((), ())),
        preferred_element_type=jnp.float32,
        precision=lax.Precision.DEFAULT,
    )

    @pl.when(k == 0)
    def _init():
        o_ref[...] = acc

    @pl.when(k > 0)
    def _acc():
        o_ref[...] += acc

    @pl.when(k == nk - 1)
    def _fin():
        o_ref[...] = jnp.maximum(o_ref[...] + b_ref[...], 0.0)


def _encode_post(xc, W_enc, b2):
    grid = (N_TOK // BM, D_SAE // BN, D_MODEL // BK)
    return pl.pallas_call(
        _mm_body,
        grid=grid,
        in_specs=[
            pl.BlockSpec((BM, BK), lambda i, j, k: (i, k)),
            pl.BlockSpec((BN, BK), lambda i, j, k: (j, k)),
            pl.BlockSpec((1, BN), lambda i, j, k: (0, j)),
        ],
        out_specs=pl.BlockSpec((BM, BN), lambda i, j, k: (i, j)),
        out_shape=jax.ShapeDtypeStruct((N_TOK, D_SAE), jnp.float32),
        compiler_params=pltpu.CompilerParams(
            dimension_semantics=("parallel", "arbitrary", "arbitrary"),
        ),
    )(xc, W_enc, b2)


NW = 32
ROWS_PER_W = N_TOK // NW
NV = D_SAE // 16
NBINS = 2048
LCAP = 64
SELW = 128
_SENT_V = -1.0
_SENT_VB = -1082130432
_SENT_I = 2**31 - 1


def _topk_select(post):
    mesh = plsc.VectorSubcoreMesh(core_axis_name="c", subcore_axis_name="s")

    @functools.partial(
        pl.kernel,
        mesh=mesh,
        compiler_params=pltpu.CompilerParams(needs_layout_passes=False),
        out_type=[
            jax.ShapeDtypeStruct((N_TOK, SELW), jnp.int32),
            jax.ShapeDtypeStruct((N_TOK, SELW), jnp.int32),
        ],
        scratch_types=[
            pltpu.VMEM((2 * D_SAE,), jnp.int32),
            pltpu.VMEM((NBINS * 16,), jnp.int32),
            pltpu.VMEM((LCAP * 16,), jnp.int32),
            pltpu.VMEM((LCAP * 16,), jnp.int32),
            pltpu.VMEM((SELW + 16,), jnp.int32),
            pltpu.VMEM((SELW + 16,), jnp.int32),
            pltpu.SemaphoreType.DMA,
            pltpu.SemaphoreType.DMA,
        ],
    )
    def sel_kernel(post_hbm, selv_hbm, seli_hbm,
                   rowbuf, hist, lanev, lanei, selv, seli, sem_in, sem_out):
        cid = lax.axis_index("c")
        sid = lax.axis_index("s")
        wid = sid * 2 + cid
        row0 = wid * ROWS_PER_W
        lanes = lax.iota(jnp.int32, 16)
        ones = jnp.ones((16,), jnp.int32)
        zero16 = jnp.zeros((16,), jnp.int32)

        def scan_down(bstart, cstart):
            def cond(st):
                b, c = st
                return c + jnp.sum(hist[pl.ds(b * 16, 16)]) < TOPK

            def body(st):
                b, c = st
                return b - 1, c + jnp.sum(hist[pl.ds(b * 16, 16)])

            return lax.while_loop(cond, body, (bstart, cstart))

        def process_row(r, base):
            @pl.when(r > row0)
            def _drain():
                pltpu.make_async_copy(
                    selv.at[pl.ds(0, SELW)], selv_hbm.at[r], sem_out).wait()
                pltpu.make_async_copy(
                    seli.at[pl.ds(0, SELW)], seli_hbm.at[r], sem_out).wait()

            def clr(j, _):
                hist[pl.ds(j * 16, 16)] = zero16
                return 0
            lax.fori_loop(0, NBINS, clr, 0, unroll=16)

            def p1(j, mx):
                u = rowbuf[pl.ds(base + j * 16, 16)]
                key = lax.shift_right_logical(u, 20)
                plsc.addupdate_scatter(hist, [key * 16 + lanes], ones)
                return jnp.maximum(mx, u)
            mx = lax.fori_loop(0, NV, p1, zero16, unroll=8)
            b1_start = lax.shift_right_logical(jnp.max(mx), 20)
            b1, c1 = scan_down(b1_start, jnp.int32(0))

            lax.fori_loop(0, NBINS, clr, 0, unroll=16)
            t1 = lax.shift_left(b1, 20)

            def p2(j, st):
                lanecnt, mx2 = st
                u = rowbuf[pl.ds(base + j * 16, 16)]
                key = lax.shift_right_logical(u, 20)
                m_bin = key == b1
                k2 = jnp.bitwise_and(lax.shift_right_logical(u, 9), 0x7FF)
                plsc.addupdate_scatter(hist, [k2 * 16 + lanes], ones, mask=m_bin)
                mx2 = jnp.maximum(mx2, jnp.where(m_bin, k2, 0))
                m_app = jnp.logical_and(u >= t1, lanecnt < LCAP)
                plsc.store_scatter(lanev, [lanecnt * 16 + lanes], u, mask=m_app)
                plsc.store_scatter(lanei, [lanecnt * 16 + lanes], j * 16 + lanes,
                                   mask=m_app)
                return lanecnt + m_app.astype(jnp.int32), mx2

            lanecnt, mx2 = lax.fori_loop(0, NV, p2, (zero16, zero16),
                                         unroll=8)
            b2, _c2 = scan_down(jnp.max(mx2), c1)
            t_lo = jnp.bitwise_or(t1, lax.shift_left(b2, 9))

            def sent(j, _):
                selv[pl.ds(j * 16, 16)] = jnp.full((16,), _SENT_VB, jnp.int32)
                seli[pl.ds(j * 16, 16)] = jnp.full((16,), _SENT_I, jnp.int32)
                return 0
            lax.fori_loop(0, (SELW + 16) // 16, sent, 0, unroll=4)

            def comp(j, cur):
                lv = lanev[pl.ds(j * 16, 16)]
                li = lanei[pl.ds(j * 16, 16)]
                m = jnp.logical_and(j < lanecnt, lv >= t_lo)
                m = jnp.logical_and(m, cur < SELW)
                plsc.store_compressed(selv.at[pl.ds(cur, 16)], lv, mask=m)
                plsc.store_compressed(seli.at[pl.ds(cur, 16)], li, mask=m)
                pc = plsc.all_reduce_population_count(m)
                return cur + jnp.max(pc)
            lax.fori_loop(0, LCAP, comp, jnp.int32(0))

            pltpu.make_async_copy(
                selv.at[pl.ds(0, SELW)], selv_hbm.at[r], sem_out).start()
            pltpu.make_async_copy(
                seli.at[pl.ds(0, SELW)], seli_hbm.at[r], sem_out).start()

        pltpu.make_async_copy(post_hbm.at[row0],
                              rowbuf.at[pl.ds(0, D_SAE)], sem_in).start()

        def outer(t, _):
            for b in (0, 1):
                i = 2 * t + b
                r = row0 + i
                pltpu.make_async_copy(
                    post_hbm.at[r], rowbuf.at[pl.ds(b * D_SAE, D_SAE)],
                    sem_in).wait()

                @pl.when(i + 1 < ROWS_PER_W)
                def _start(r=r, b=b):
                    pltpu.make_async_copy(
                        post_hbm.at[r + 1],
                        rowbuf.at[pl.ds((1 - b) * D_SAE, D_SAE)],
                        sem_in).start()

                process_row(r, b * D_SAE)
            return 0

        lax.fori_loop(0, ROWS_PER_W // 2, outer, 0)
        rlast = row0 + ROWS_PER_W - 1
        pltpu.make_async_copy(
            selv.at[pl.ds(0, SELW)], selv_hbm.at[rlast], sem_out).wait()
        pltpu.make_async_copy(
            seli.at[pl.ds(0, SELW)], seli_hbm.at[rlast], sem_out).wait()

    return sel_kernel(post)


BR = 512
BC = 2048


def _rank_body(selv_ref, seli_ref, post_ref, ta_ref, ti_ref, enc_ref, t_ref):
    j = pl.program_id(1)

    @pl.when(j == 0)
    def _sort():
        v = selv_ref[...]
        ix = seli_ref[...]
        iota = lax.broadcasted_iota(jnp.int32, (BR, SELW), 1)
        for s in range(7):
            asc = (iota & (2 << s)) == 0
            for d in [1 << t for t in range(s, -1, -1)]:
                upper = (iota & d) != 0
                pv = jnp.where(upper, pltpu.roll(v, d, 1),
                               pltpu.roll(v, SELW - d, 1))
                pi = jnp.where(upper, pltpu.roll(ix, d, 1),
                               pltpu.roll(ix, SELW - d, 1))
                self_first = (v > pv) | ((v == pv) & (ix < pi))
                take_max = upper == asc
                takes_self = jnp.logical_xor(self_first, take_max)
                v = jnp.where(takes_self, v, pv)
                ix = jnp.where(takes_self, ix, pi)
        ta_ref[...] = v
        ti_ref[...] = ix
        t_ref[...] = jnp.broadcast_to(v[:, (TOPK - 1):TOPK], (BR, SELW))

    enc_ref[...] = jnp.where(post_ref[...] >= t_ref[:, 0:1], post_ref[...],
                             0.0)


def _rank_and_mask(selv, seli, post):
    grid = (N_TOK // BR, D_SAE // BC)
    return pl.pallas_call(
        _rank_body,
        grid=grid,
        in_specs=[
            pl.BlockSpec((BR, SELW), lambda i, j: (i, 0)),
            pl.BlockSpec((BR, SELW), lambda i, j: (i, 0)),
            pl.BlockSpec((BR, BC), lambda i, j: (i, j)),
        ],
        out_specs=[
            pl.BlockSpec((BR, SELW), lambda i, j: (i, 0)),
            pl.BlockSpec((BR, SELW), lambda i, j: (i, 0)),
            pl.BlockSpec((BR, BC), lambda i, j: (i, j)),
        ],
        out_shape=[
            jax.ShapeDtypeStruct((N_TOK, SELW), jnp.float32),
            jax.ShapeDtypeStruct((N_TOK, SELW), jnp.int32),
            jax.ShapeDtypeStruct((N_TOK, D_SAE), jnp.float32),
        ],
        scratch_shapes=[pltpu.VMEM((BR, SELW), jnp.float32)],
        compiler_params=pltpu.CompilerParams(
            dimension_semantics=("parallel", "arbitrary"),
        ),
    )(selv, seli, post)


def kernel(x, W_enc, b_enc, dec_bias):
    xc = x - dec_bias[None, :]
    post = _encode_post(xc, W_enc, b_enc.reshape(1, -1))
    post_bits = lax.bitcast_convert_type(post, jnp.int32)
    selv_b, seli = _topk_select(post_bits)
    selv = lax.bitcast_convert_type(selv_b, jnp.float32)
    top_v, top_i, encoded_acts = _rank_and_mask(selv, seli, post)
    return encoded_acts, top_v[:, :TOPK], top_i[:, :TOPK]


# speedup vs baseline: 6.1670x; 1.4281x over previous
import functools

import jax
import jax.numpy as jnp
from jax import lax
from jax.experimental import pallas as pl
from jax.experimental.pallas import tpu as pltpu
from jax.experimental.pallas import tpu_sc as plsc

D_MODEL = 4096
D_SAE = 32768
N_TOK = 4096
TOPK = 100


BM = 1024
BN = 1024
BK = 512


def _mm_body(x_ref, w_ref, b_ref, o_ref):
    k = pl.program_id(2)
    nk = pl.num_programs(2)
    acc = lax.dot_general(
        x_ref[...], w_ref[...],
        (((1,), (1,)), ((), ())),
        preferred_element_type=jnp.float32,
        precision=lax.Precision.DEFAULT,
    )

    @pl.when(k == 0)
    def _init():
        o_ref[...] = acc

    @pl.when(k > 0)
    def _acc():
        o_ref[...] += acc

    @pl.when(k == nk - 1)
    def _fin():
        o_ref[...] = jnp.maximum(o_ref[...] + b_ref[...], 0.0)


def _encode_post(xc, W_enc, b2):
    grid = (N_TOK // BM, D_SAE // BN, D_MODEL // BK)
    return pl.pallas_call(
        _mm_body,
        grid=grid,
        in_specs=[
            pl.BlockSpec((BM, BK), lambda i, j, k: (i, k)),
            pl.BlockSpec((BN, BK), lambda i, j, k: (j, k)),
            pl.BlockSpec((1, BN), lambda i, j, k: (0, j)),
        ],
        out_specs=pl.BlockSpec((BM, BN), lambda i, j, k: (i, j)),
        out_shape=jax.ShapeDtypeStruct((N_TOK, D_SAE), jnp.float32),
        compiler_params=pltpu.CompilerParams(
            dimension_semantics=("parallel", "arbitrary", "arbitrary"),
        ),
    )(xc, W_enc, b2)


NW = 32
ROWS_PER_W = N_TOK // NW
NV = D_SAE // 16
NBINS = 2048
LCAP = 64
SELW = 128
_SENT_V = -1.0
_SENT_VB = -1082130432
_SENT_I = 2**31 - 1


def _topk_select(post):
    mesh = plsc.VectorSubcoreMesh(core_axis_name="c", subcore_axis_name="s")

    @functools.partial(
        pl.kernel,
        mesh=mesh,
        compiler_params=pltpu.CompilerParams(needs_layout_passes=False),
        out_type=[
            jax.ShapeDtypeStruct((N_TOK, SELW), jnp.int32),
            jax.ShapeDtypeStruct((N_TOK, SELW), jnp.int32),
        ],
        scratch_types=[
            pltpu.VMEM((2 * D_SAE,), jnp.int32),
            pltpu.VMEM((NBINS * 16,), jnp.int32),
            pltpu.VMEM((LCAP * 16,), jnp.int32),
            pltpu.VMEM((LCAP * 16,), jnp.int32),
            pltpu.VMEM((SELW + 16,), jnp.int32),
            pltpu.VMEM((SELW + 16,), jnp.int32),
            pltpu.SemaphoreType.DMA,
            pltpu.SemaphoreType.DMA,
        ],
    )
    def sel_kernel(post_hbm, selv_hbm, seli_hbm,
                   rowbuf, hist, lanev, lanei, selv, seli, sem_in, sem_out):
        cid = lax.axis_index("c")
        sid = lax.axis_index("s")
        wid = sid * 2 + cid
        row0 = wid * ROWS_PER_W
        lanes = lax.iota(jnp.int32, 16)
        ones = jnp.ones((16,), jnp.int32)
        zero16 = jnp.zeros((16,), jnp.int32)

        def scan_down(bstart, cstart):
            def cond(st):
                b, c = st
                return c + jnp.sum(hist[pl.ds(b * 16, 16)]) < TOPK

            def body(st):
                b, c = st
                return b - 1, c + jnp.sum(hist[pl.ds(b * 16, 16)])

            return lax.while_loop(cond, body, (bstart, cstart))

        def process_row(r, base):
            @pl.when(r > row0)
            def _drain():
                pltpu.make_async_copy(
                    selv.at[pl.ds(0, SELW)], selv_hbm.at[r], sem_out).wait()
                pltpu.make_async_copy(
                    seli.at[pl.ds(0, SELW)], seli_hbm.at[r], sem_out).wait()

            def clr(j):
                hist[pl.ds(j * 16, 16)] = zero16
            plsc.parallel_loop(0, NBINS, unroll=8)(clr)

            def p1(j, mx):
                u = rowbuf[pl.ds(base + j * 16, 16)]
                key = lax.shift_right_logical(u, 20)
                plsc.addupdate_scatter(hist, [key * 16 + lanes], ones)
                return jnp.maximum(mx, u)
            mx = plsc.parallel_loop(0, NV, unroll=8, carry=zero16)(p1)
            b1_start = lax.shift_right_logical(jnp.max(mx), 20)
            b1, c1 = scan_down(b1_start, jnp.int32(0))

            plsc.parallel_loop(0, NBINS, unroll=8)(clr)
            t1 = lax.shift_left(b1, 20)

            def p2(j, st):
                lanecnt, mx2 = st
                u = rowbuf[pl.ds(base + j * 16, 16)]
                key = lax.shift_right_logical(u, 20)
                m_bin = key == b1
                k2 = jnp.bitwise_and(lax.shift_right_logical(u, 9), 0x7FF)
                plsc.addupdate_scatter(hist, [k2 * 16 + lanes], ones, mask=m_bin)
                mx2 = jnp.maximum(mx2, jnp.where(m_bin, k2, 0))
                m_app = jnp.logical_and(u >= t1, lanecnt < LCAP)
                plsc.store_scatter(lanev, [lanecnt * 16 + lanes], u, mask=m_app)
                plsc.store_scatter(lanei, [lanecnt * 16 + lanes], j * 16 + lanes,
                                   mask=m_app)
                return lanecnt + m_app.astype(jnp.int32), mx2

            lanecnt, mx2 = plsc.parallel_loop(
                0, NV, unroll=8, carry=(zero16, zero16))(p2)
            b2, _c2 = scan_down(jnp.max(mx2), c1)
            t_lo = jnp.bitwise_or(t1, lax.shift_left(b2, 9))

            def sent(j):
                selv[pl.ds(j * 16, 16)] = jnp.full((16,), _SENT_VB, jnp.int32)
                seli[pl.ds(j * 16, 16)] = jnp.full((16,), _SENT_I, jnp.int32)
            plsc.parallel_loop(0, (SELW + 16) // 16, unroll=4)(sent)

            def comp(j, cur):
                lv = lanev[pl.ds(j * 16, 16)]
                li = lanei[pl.ds(j * 16, 16)]
                m = jnp.logical_and(j < lanecnt, lv >= t_lo)
                m = jnp.logical_and(m, cur < SELW)
                plsc.store_compressed(selv.at[pl.ds(cur, 16)], lv, mask=m)
                plsc.store_compressed(seli.at[pl.ds(cur, 16)], li, mask=m)
                pc = plsc.all_reduce_population_count(m)
                return cur + jnp.max(pc)
            plsc.parallel_loop(0, LCAP, carry=jnp.int32(0))(comp)

            pltpu.make_async_copy(
                selv.at[pl.ds(0, SELW)], selv_hbm.at[r], sem_out).start()
            pltpu.make_async_copy(
                seli.at[pl.ds(0, SELW)], seli_hbm.at[r], sem_out).start()

        pltpu.make_async_copy(post_hbm.at[row0],
                              rowbuf.at[pl.ds(0, D_SAE)], sem_in).start()

        def outer(t, _):
            for b in (0, 1):
                i = 2 * t + b
                r = row0 + i
                pltpu.make_async_copy(
                    post_hbm.at[r], rowbuf.at[pl.ds(b * D_SAE, D_SAE)],
                    sem_in).wait()

                @pl.when(i + 1 < ROWS_PER_W)
                def _start(r=r, b=b):
                    pltpu.make_async_copy(
                        post_hbm.at[r + 1],
                        rowbuf.at[pl.ds((1 - b) * D_SAE, D_SAE)],
                        sem_in).start()

                process_row(r, b * D_SAE)
            return 0

        lax.fori_loop(0, ROWS_PER_W // 2, outer, 0)
        rlast = row0 + ROWS_PER_W - 1
        pltpu.make_async_copy(
            selv.at[pl.ds(0, SELW)], selv_hbm.at[rlast], sem_out).wait()
        pltpu.make_async_copy(
            seli.at[pl.ds(0, SELW)], seli_hbm.at[rlast], sem_out).wait()

    return sel_kernel(post)


BR = 512
BC = 2048


def _rank_body(selv_ref, seli_ref, post_ref, ta_ref, ti_ref, enc_ref, t_ref):
    j = pl.program_id(1)

    @pl.when(j == 0)
    def _sort():
        v = selv_ref[...]
        ix = seli_ref[...]
        iota = lax.broadcasted_iota(jnp.int32, (BR, SELW), 1)
        for s in range(7):
            asc = (iota & (2 << s)) == 0
            for d in [1 << t for t in range(s, -1, -1)]:
                upper = (iota & d) != 0
                pv = jnp.where(upper, pltpu.roll(v, d, 1),
                               pltpu.roll(v, SELW - d, 1))
                pi = jnp.where(upper, pltpu.roll(ix, d, 1),
                               pltpu.roll(ix, SELW - d, 1))
                self_first = (v > pv) | ((v == pv) & (ix < pi))
                take_max = upper == asc
                takes_self = jnp.logical_xor(self_first, take_max)
                v = jnp.where(takes_self, v, pv)
                ix = jnp.where(takes_self, ix, pi)
        ta_ref[...] = v
        ti_ref[...] = ix
        t_ref[...] = jnp.broadcast_to(v[:, (TOPK - 1):TOPK], (BR, SELW))

    enc_ref[...] = jnp.where(post_ref[...] >= t_ref[:, 0:1], post_ref[...],
                             0.0)


def _rank_and_mask(selv, seli, post):
    grid = (N_TOK // BR, D_SAE // BC)
    return pl.pallas_call(
        _rank_body,
        grid=grid,
        in_specs=[
            pl.BlockSpec((BR, SELW), lambda i, j: (i, 0)),
            pl.BlockSpec((BR, SELW), lambda i, j: (i, 0)),
            pl.BlockSpec((BR, BC), lambda i, j: (i, j)),
        ],
        out_specs=[
            pl.BlockSpec((BR, SELW), lambda i, j: (i, 0)),
            pl.BlockSpec((BR, SELW), lambda i, j: (i, 0)),
            pl.BlockSpec((BR, BC), lambda i, j: (i, j)),
        ],
        out_shape=[
            jax.ShapeDtypeStruct((N_TOK, SELW), jnp.float32),
            jax.ShapeDtypeStruct((N_TOK, SELW), jnp.int32),
            jax.ShapeDtypeStruct((N_TOK, D_SAE), jnp.float32),
        ],
        scratch_shapes=[pltpu.VMEM((BR, SELW), jnp.float32)],
        compiler_params=pltpu.CompilerParams(
            dimension_semantics=("parallel", "arbitrary"),
        ),
    )(selv, seli, post)


def kernel(x, W_enc, b_enc, dec_bias):
    xc = x - dec_bias[None, :]
    post = _encode_post(xc, W_enc, b_enc.reshape(1, -1))
    post_bits = lax.bitcast_convert_type(post, jnp.int32)
    selv_b, seli = _topk_select(post_bits)
    selv = lax.bitcast_convert_type(selv_b, jnp.float32)
    top_v, top_i, encoded_acts = _rank_and_mask(selv, seli, post)
    return encoded_acts, top_v[:, :TOPK], top_i[:, :TOPK]


# speedup vs baseline: 7.1777x; 1.1639x over previous
import functools

import jax
import jax.numpy as jnp
from jax import lax
from jax.experimental import pallas as pl
from jax.experimental.pallas import tpu as pltpu
from jax.experimental.pallas import tpu_sc as plsc

D_MODEL = 4096
D_SAE = 32768
N_TOK = 4096
TOPK = 100


BM = 1024
BN = 1024
BK = 512


def _mm_body(x_ref, w_ref, b_ref, o_ref):
    k = pl.program_id(2)
    nk = pl.num_programs(2)
    acc = lax.dot_general(
        x_ref[...], w_ref[...],
        (((1,), (1,)), ((), ())),
        preferred_element_type=jnp.float32,
        precision=lax.Precision.DEFAULT,
    )

    @pl.when(k == 0)
    def _init():
        o_ref[...] = acc

    @pl.when(k > 0)
    def _acc():
        o_ref[...] += acc

    @pl.when(k == nk - 1)
    def _fin():
        o_ref[...] = jnp.maximum(o_ref[...] + b_ref[...], 0.0)


def _encode_post(xc, W_enc, b2):
    m_tok = xc.shape[0]
    grid = (m_tok // BM, D_SAE // BN, D_MODEL // BK)
    return pl.pallas_call(
        _mm_body,
        grid=grid,
        in_specs=[
            pl.BlockSpec((BM, BK), lambda i, j, k: (i, k)),
            pl.BlockSpec((BN, BK), lambda i, j, k: (j, k)),
            pl.BlockSpec((1, BN), lambda i, j, k: (0, j)),
        ],
        out_specs=pl.BlockSpec((BM, BN), lambda i, j, k: (i, j)),
        out_shape=jax.ShapeDtypeStruct((m_tok, D_SAE), jnp.float32),
        compiler_params=pltpu.CompilerParams(
            dimension_semantics=("parallel", "arbitrary", "arbitrary"),
        ),
    )(xc, W_enc, b2)


NW = 32
ROWS_PER_W = N_TOK // NW
NV = D_SAE // 16
NBINS = 2048
LCAP = 64
SELW = 128
_SENT_V = -1.0
_SENT_VB = -1082130432
_SENT_I = 2**31 - 1


def _topk_select(post):
    m_tok = post.shape[0]
    rows_per_w = m_tok // NW
    mesh = plsc.VectorSubcoreMesh(core_axis_name="c", subcore_axis_name="s")

    @functools.partial(
        pl.kernel,
        mesh=mesh,
        compiler_params=pltpu.CompilerParams(needs_layout_passes=False),
        out_type=[
            jax.ShapeDtypeStruct((m_tok, SELW), jnp.int32),
            jax.ShapeDtypeStruct((m_tok, SELW), jnp.int32),
        ],
        scratch_types=[
            pltpu.VMEM((2 * D_SAE,), jnp.int32),
            pltpu.VMEM((NBINS * 16,), jnp.int32),
            pltpu.VMEM((LCAP * 16,), jnp.int32),
            pltpu.VMEM((LCAP * 16,), jnp.int32),
            pltpu.VMEM((SELW + 16,), jnp.int32),
            pltpu.VMEM((SELW + 16,), jnp.int32),
            pltpu.SemaphoreType.DMA,
            pltpu.SemaphoreType.DMA,
        ],
    )
    def sel_kernel(post_hbm, selv_hbm, seli_hbm,
                   rowbuf, hist, lanev, lanei, selv, seli, sem_in, sem_out):
        cid = lax.axis_index("c")
        sid = lax.axis_index("s")
        wid = sid * 2 + cid
        row0 = wid * rows_per_w
        lanes = lax.iota(jnp.int32, 16)
        ones = jnp.ones((16,), jnp.int32)
        zero16 = jnp.zeros((16,), jnp.int32)

        def scan_down(bstart, cstart):
            def cond(st):
                b, c = st
                return c + jnp.sum(hist[pl.ds(b * 16, 16)]) < TOPK

            def body(st):
                b, c = st
                return b - 1, c + jnp.sum(hist[pl.ds(b * 16, 16)])

            return lax.while_loop(cond, body, (bstart, cstart))

        def process_row(r, base):
            @pl.when(r > row0)
            def _drain():
                pltpu.make_async_copy(
                    selv.at[pl.ds(0, SELW)], selv_hbm.at[r], sem_out).wait()
                pltpu.make_async_copy(
                    seli.at[pl.ds(0, SELW)], seli_hbm.at[r], sem_out).wait()

            def clr(j):
                hist[pl.ds(j * 16, 16)] = zero16
            plsc.parallel_loop(0, NBINS, unroll=8)(clr)

            def p1(j, mx):
                u = rowbuf[pl.ds(base + j * 16, 16)]
                key = lax.shift_right_logical(u, 20)
                plsc.addupdate_scatter(hist, [key * 16 + lanes], ones)
                return jnp.maximum(mx, u)
            mx = plsc.parallel_loop(0, NV, unroll=8, carry=zero16)(p1)
            b1_start = lax.shift_right_logical(jnp.max(mx), 20)
            b1, c1 = scan_down(b1_start, jnp.int32(0))

            plsc.parallel_loop(0, NBINS, unroll=8)(clr)
            t1 = lax.shift_left(b1, 20)

            def p2(j, st):
                lanecnt, mx2 = st
                u = rowbuf[pl.ds(base + j * 16, 16)]
                key = lax.shift_right_logical(u, 20)
                m_bin = key == b1
                k2 = jnp.bitwise_and(lax.shift_right_logical(u, 9), 0x7FF)
                plsc.addupdate_scatter(hist, [k2 * 16 + lanes], ones, mask=m_bin)
                mx2 = jnp.maximum(mx2, jnp.where(m_bin, k2, 0))
                m_app = jnp.logical_and(u >= t1, lanecnt < LCAP)
                plsc.store_scatter(lanev, [lanecnt * 16 + lanes], u, mask=m_app)
                plsc.store_scatter(lanei, [lanecnt * 16 + lanes], j * 16 + lanes,
                                   mask=m_app)
                return lanecnt + m_app.astype(jnp.int32), mx2

            lanecnt, mx2 = plsc.parallel_loop(
                0, NV, unroll=8, carry=(zero16, zero16))(p2)
            b2, _c2 = scan_down(jnp.max(mx2), c1)
            t_lo = jnp.bitwise_or(t1, lax.shift_left(b2, 9))

            def sent(j):
                selv[pl.ds(j * 16, 16)] = jnp.full((16,), _SENT_VB, jnp.int32)
                seli[pl.ds(j * 16, 16)] = jnp.full((16,), _SENT_I, jnp.int32)
            plsc.parallel_loop(0, (SELW + 16) // 16, unroll=4)(sent)

            def comp(j, cur):
                lv = lanev[pl.ds(j * 16, 16)]
                li = lanei[pl.ds(j * 16, 16)]
                m = jnp.logical_and(j < lanecnt, lv >= t_lo)
                m = jnp.logical_and(m, cur < SELW)
                plsc.store_compressed(selv.at[pl.ds(cur, 16)], lv, mask=m)
                plsc.store_compressed(seli.at[pl.ds(cur, 16)], li, mask=m)
                pc = plsc.all_reduce_population_count(m)
                return cur + jnp.max(pc)
            plsc.parallel_loop(0, LCAP, carry=jnp.int32(0))(comp)

            pltpu.make_async_copy(
                selv.at[pl.ds(0, SELW)], selv_hbm.at[r], sem_out).start()
            pltpu.make_async_copy(
                seli.at[pl.ds(0, SELW)], seli_hbm.at[r], sem_out).start()

        pltpu.make_async_copy(post_hbm.at[row0],
                              rowbuf.at[pl.ds(0, D_SAE)], sem_in).start()

        def outer(t, _):
            for b in (0, 1):
                i = 2 * t + b
                r = row0 + i
                pltpu.make_async_copy(
                    post_hbm.at[r], rowbuf.at[pl.ds(b * D_SAE, D_SAE)],
                    sem_in).wait()

                @pl.when(i + 1 < rows_per_w)
                def _start(r=r, b=b):
                    pltpu.make_async_copy(
                        post_hbm.at[r + 1],
                        rowbuf.at[pl.ds((1 - b) * D_SAE, D_SAE)],
                        sem_in).start()

                process_row(r, b * D_SAE)
            return 0

        lax.fori_loop(0, rows_per_w // 2, outer, 0)
        rlast = row0 + rows_per_w - 1
        pltpu.make_async_copy(
            selv.at[pl.ds(0, SELW)], selv_hbm.at[rlast], sem_out).wait()
        pltpu.make_async_copy(
            seli.at[pl.ds(0, SELW)], seli_hbm.at[rlast], sem_out).wait()

    return sel_kernel(post)


BR = 512
BC = 2048


def _rank_body(selv_ref, seli_ref, post_ref, *rest):
    ta_ref, ti_ref, enc_ref, t_ref = rest[-4:]
    j = pl.program_id(1)

    @pl.when(j == 0)
    def _sort():
        v = selv_ref[...]
        ix = seli_ref[...]
        iota = lax.broadcasted_iota(jnp.int32, (BR, SELW), 1)
        for s in range(7):
            asc = (iota & (2 << s)) == 0
            for d in [1 << t for t in range(s, -1, -1)]:
                upper = (iota & d) != 0
                pv = jnp.where(upper, pltpu.roll(v, d, 1),
                               pltpu.roll(v, SELW - d, 1))
                pi = jnp.where(upper, pltpu.roll(ix, d, 1),
                               pltpu.roll(ix, SELW - d, 1))
                self_first = (v > pv) | ((v == pv) & (ix < pi))
                take_max = upper == asc
                takes_self = jnp.logical_xor(self_first, take_max)
                v = jnp.where(takes_self, v, pv)
                ix = jnp.where(takes_self, ix, pi)
        ta_ref[...] = v
        ti_ref[...] = ix
        t_ref[...] = jnp.broadcast_to(v[:, (TOPK - 1):TOPK], (BR, SELW))

    enc_ref[...] = jnp.where(post_ref[...] >= t_ref[:, 0:1], post_ref[...],
                             0.0)


def _rank_and_mask(selv, seli, post, row_off, enc_prev=None):
    m_tok = post.shape[0]
    grid = (m_tok // BR, D_SAE // BC)
    ro = row_off // BR
    in_specs = [
        pl.BlockSpec((BR, SELW), lambda i, j: (i, 0)),
        pl.BlockSpec((BR, SELW), lambda i, j: (i, 0)),
        pl.BlockSpec((BR, BC), lambda i, j: (i, j)),
    ]
    args = [selv, seli, post]
    aliases = {}
    if enc_prev is not None:
        in_specs.append(pl.BlockSpec(memory_space=pl.ANY))
        args.append(enc_prev)
        aliases = {3: 2}
    return pl.pallas_call(
        _rank_body,
        grid=grid,
        in_specs=in_specs,
        out_specs=[
            pl.BlockSpec((BR, SELW), lambda i, j: (i, 0)),
            pl.BlockSpec((BR, SELW), lambda i, j: (i, 0)),
            pl.BlockSpec((BR, BC), lambda i, j: (i + ro, j)),
        ],
        out_shape=[
            jax.ShapeDtypeStruct((m_tok, SELW), jnp.float32),
            jax.ShapeDtypeStruct((m_tok, SELW), jnp.int32),
            jax.ShapeDtypeStruct((N_TOK, D_SAE), jnp.float32),
        ],
        scratch_shapes=[pltpu.VMEM((BR, SELW), jnp.float32)],
        input_output_aliases=aliases,
        compiler_params=pltpu.CompilerParams(
            dimension_semantics=("parallel", "arbitrary"),
        ),
    )(*args)


def kernel(x, W_enc, b_enc, dec_bias):
    xc = x - dec_bias[None, :]
    b2 = b_enc.reshape(1, -1)
    half = N_TOK // 2
    tops = []
    enc = None
    posts = [_encode_post(xc[h * half:(h + 1) * half], W_enc, b2)
             for h in (0, 1)]
    sels = [_topk_select(lax.bitcast_convert_type(p, jnp.int32))
            for p in posts]
    for h in (0, 1):
        sv, si = sels[h]
        tv, ti, enc = _rank_and_mask(
            lax.bitcast_convert_type(sv, jnp.float32), si, posts[h],
            h * half, enc)
        tops.append((tv, ti))
    top_v = jnp.concatenate([tops[0][0], tops[1][0]], axis=0)
    top_i = jnp.concatenate([tops[0][1], tops[1][1]], axis=0)
    return enc, top_v[:, :TOPK], top_i[:, :TOPK]


# speedup vs baseline: 7.5029x; 1.0453x over previous
import functools

import jax
import jax.numpy as jnp
from jax import lax
from jax.experimental import pallas as pl
from jax.experimental.pallas import tpu as pltpu
from jax.experimental.pallas import tpu_sc as plsc

D_MODEL = 4096
D_SAE = 32768
N_TOK = 4096
TOPK = 100


BM = 1024
BN = 1024
BK = 512


def _mm_body(x_ref, w_ref, b_ref, o_ref):
    k = pl.program_id(2)
    nk = pl.num_programs(2)
    acc = lax.dot_general(
        x_ref[...], w_ref[...],
        (((1,), (1,)), ((), ())),
        preferred_element_type=jnp.float32,
        precision=lax.Precision.DEFAULT,
    )

    @pl.when(k == 0)
    def _init():
        o_ref[...] = acc

    @pl.when(k > 0)
    def _acc():
        o_ref[...] += acc

    @pl.when(k == nk - 1)
    def _fin():
        o_ref[...] = jnp.maximum(o_ref[...] + b_ref[...], 0.0)


def _encode_post(xc, W_enc, b2):
    m_tok = xc.shape[0]
    grid = (m_tok // BM, D_SAE // BN, D_MODEL // BK)
    return pl.pallas_call(
        _mm_body,
        grid=grid,
        in_specs=[
            pl.BlockSpec((BM, BK), lambda i, j, k: (i, k)),
            pl.BlockSpec((BN, BK), lambda i, j, k: (j, k)),
            pl.BlockSpec((1, BN), lambda i, j, k: (0, j)),
        ],
        out_specs=pl.BlockSpec((BM, BN), lambda i, j, k: (i, j)),
        out_shape=jax.ShapeDtypeStruct((m_tok, D_SAE), jnp.float32),
        compiler_params=pltpu.CompilerParams(
            dimension_semantics=("parallel", "arbitrary", "arbitrary"),
        ),
    )(xc, W_enc, b2)


NW = 32
ROWS_PER_W = N_TOK // NW
NV = D_SAE // 16
NBINS = 2048
LCAP = 64
SELW = 128
_SENT_V = -1.0
_SENT_VB = -1082130432
_SENT_I = 2**31 - 1


def _topk_select(post):
    m_tok = post.shape[0]
    rows_per_w = m_tok // NW
    mesh = plsc.VectorSubcoreMesh(core_axis_name="c", subcore_axis_name="s")

    @functools.partial(
        pl.kernel,
        mesh=mesh,
        compiler_params=pltpu.CompilerParams(needs_layout_passes=False),
        out_type=[
            jax.ShapeDtypeStruct((m_tok, SELW), jnp.int32),
            jax.ShapeDtypeStruct((m_tok, SELW), jnp.int32),
        ],
        scratch_types=[
            pltpu.VMEM((2 * D_SAE,), jnp.int32),
            pltpu.VMEM((NBINS * 16,), jnp.int32),
            pltpu.VMEM((LCAP * 16,), jnp.int32),
            pltpu.VMEM((LCAP * 16,), jnp.int32),
            pltpu.VMEM((SELW + 16,), jnp.int32),
            pltpu.VMEM((SELW + 16,), jnp.int32),
            pltpu.SemaphoreType.DMA,
            pltpu.SemaphoreType.DMA,
        ],
    )
    def sel_kernel(post_hbm, selv_hbm, seli_hbm,
                   rowbuf, hist, lanev, lanei, selv, seli, sem_in, sem_out):
        cid = lax.axis_index("c")
        sid = lax.axis_index("s")
        wid = sid * 2 + cid
        row0 = wid * rows_per_w
        lanes = lax.iota(jnp.int32, 16)
        ones = jnp.ones((16,), jnp.int32)
        zero16 = jnp.zeros((16,), jnp.int32)

        def scan_down(bstart, cstart):
            def cond(st):
                b, c = st
                return c + jnp.sum(hist[pl.ds(b * 16, 16)]) < TOPK

            def body(st):
                b, c = st
                return b - 1, c + jnp.sum(hist[pl.ds(b * 16, 16)])

            return lax.while_loop(cond, body, (bstart, cstart))

        def process_row(r, base):
            @pl.when(r > row0)
            def _drain():
                pltpu.make_async_copy(
                    selv.at[pl.ds(0, SELW)], selv_hbm.at[r], sem_out).wait()
                pltpu.make_async_copy(
                    seli.at[pl.ds(0, SELW)], seli_hbm.at[r], sem_out).wait()

            def clr(j):
                hist[pl.ds(j * 16, 16)] = zero16
            plsc.parallel_loop(0, NBINS, unroll=8)(clr)

            def p1(j, mx):
                u = rowbuf[pl.ds(base + j * 16, 16)]
                key = lax.shift_right_logical(u, 20)
                plsc.addupdate_scatter(hist, [key * 16 + lanes], ones)
                return jnp.maximum(mx, u)
            mx = plsc.parallel_loop(0, NV, unroll=8, carry=zero16)(p1)
            b1_start = lax.shift_right_logical(jnp.max(mx), 20)
            b1, c1 = scan_down(b1_start, jnp.int32(0))

            plsc.parallel_loop(0, NBINS, unroll=8)(clr)
            t1 = lax.shift_left(b1, 20)

            def p2(j, st):
                lanecnt, mx2 = st
                u = rowbuf[pl.ds(base + j * 16, 16)]
                key = lax.shift_right_logical(u, 20)
                m_bin = key == b1
                k2 = jnp.bitwise_and(lax.shift_right_logical(u, 9), 0x7FF)
                plsc.addupdate_scatter(hist, [k2 * 16 + lanes], ones, mask=m_bin)
                mx2 = jnp.maximum(mx2, jnp.where(m_bin, k2, 0))
                m_app = jnp.logical_and(u >= t1, lanecnt < LCAP)
                plsc.store_scatter(lanev, [lanecnt * 16 + lanes], u, mask=m_app)
                plsc.store_scatter(lanei, [lanecnt * 16 + lanes], j * 16 + lanes,
                                   mask=m_app)
                return lanecnt + m_app.astype(jnp.int32), mx2

            lanecnt, mx2 = plsc.parallel_loop(
                0, NV, unroll=8, carry=(zero16, zero16))(p2)
            b2, _c2 = scan_down(jnp.max(mx2), c1)
            t_lo = jnp.bitwise_or(t1, lax.shift_left(b2, 9))

            def sent(j):
                selv[pl.ds(j * 16, 16)] = jnp.full((16,), _SENT_VB, jnp.int32)
                seli[pl.ds(j * 16, 16)] = jnp.full((16,), _SENT_I, jnp.int32)
            plsc.parallel_loop(0, (SELW + 16) // 16, unroll=4)(sent)

            def comp(j, cur):
                lv = lanev[pl.ds(j * 16, 16)]
                li = lanei[pl.ds(j * 16, 16)]
                m = jnp.logical_and(j < lanecnt, lv >= t_lo)
                m = jnp.logical_and(m, cur < SELW)
                plsc.store_compressed(selv.at[pl.ds(cur, 16)], lv, mask=m)
                plsc.store_compressed(seli.at[pl.ds(cur, 16)], li, mask=m)
                pc = plsc.all_reduce_population_count(m)
                return cur + jnp.max(pc)
            plsc.parallel_loop(0, LCAP, carry=jnp.int32(0))(comp)

            pltpu.make_async_copy(
                selv.at[pl.ds(0, SELW)], selv_hbm.at[r], sem_out).start()
            pltpu.make_async_copy(
                seli.at[pl.ds(0, SELW)], seli_hbm.at[r], sem_out).start()

        pltpu.make_async_copy(post_hbm.at[row0],
                              rowbuf.at[pl.ds(0, D_SAE)], sem_in).start()

        def outer(t, _):
            for b in (0, 1):
                i = 2 * t + b
                r = row0 + i
                pltpu.make_async_copy(
                    post_hbm.at[r], rowbuf.at[pl.ds(b * D_SAE, D_SAE)],
                    sem_in).wait()

                @pl.when(i + 1 < rows_per_w)
                def _start(r=r, b=b):
                    pltpu.make_async_copy(
                        post_hbm.at[r + 1],
                        rowbuf.at[pl.ds((1 - b) * D_SAE, D_SAE)],
                        sem_in).start()

                process_row(r, b * D_SAE)
            return 0

        lax.fori_loop(0, rows_per_w // 2, outer, 0)
        rlast = row0 + rows_per_w - 1
        pltpu.make_async_copy(
            selv.at[pl.ds(0, SELW)], selv_hbm.at[rlast], sem_out).wait()
        pltpu.make_async_copy(
            seli.at[pl.ds(0, SELW)], seli_hbm.at[rlast], sem_out).wait()

    return sel_kernel(post)


BR = 512
BC = 2048


def _rank_body(selv_ref, seli_ref, post_ref, *rest):
    ta_ref, ti_ref, enc_ref, t_ref = rest[-4:]
    j = pl.program_id(1)

    @pl.when(j == 0)
    def _sort():
        v = selv_ref[...]
        ix = seli_ref[...]
        iota = lax.broadcasted_iota(jnp.int32, (BR, SELW), 1)
        for s in range(7):
            asc = (iota & (2 << s)) == 0
            for d in [1 << t for t in range(s, -1, -1)]:
                upper = (iota & d) != 0
                pv = jnp.where(upper, pltpu.roll(v, d, 1),
                               pltpu.roll(v, SELW - d, 1))
                pi = jnp.where(upper, pltpu.roll(ix, d, 1),
                               pltpu.roll(ix, SELW - d, 1))
                self_first = (v > pv) | ((v == pv) & (ix < pi))
                take_max = upper == asc
                takes_self = jnp.logical_xor(self_first, take_max)
                v = jnp.where(takes_self, v, pv)
                ix = jnp.where(takes_self, ix, pi)
        ta_ref[...] = v
        ti_ref[...] = ix
        t_ref[...] = jnp.broadcast_to(v[:, (TOPK - 1):TOPK], (BR, SELW))

    enc_ref[...] = jnp.where(post_ref[...] >= t_ref[:, 0:1], post_ref[...],
                             0.0)


def _rank_and_mask(selv, seli, post, row_off, enc_prev=None):
    m_tok = post.shape[0]
    grid = (m_tok // BR, D_SAE // BC)
    ro = row_off // BR
    in_specs = [
        pl.BlockSpec((BR, SELW), lambda i, j: (i, 0)),
        pl.BlockSpec((BR, SELW), lambda i, j: (i, 0)),
        pl.BlockSpec((BR, BC), lambda i, j: (i, j)),
    ]
    args = [selv, seli, post]
    aliases = {}
    if enc_prev is not None:
        in_specs.append(pl.BlockSpec(memory_space=pl.ANY))
        args.append(enc_prev)
        aliases = {3: 2}
    return pl.pallas_call(
        _rank_body,
        grid=grid,
        in_specs=in_specs,
        out_specs=[
            pl.BlockSpec((BR, SELW), lambda i, j: (i, 0)),
            pl.BlockSpec((BR, SELW), lambda i, j: (i, 0)),
            pl.BlockSpec((BR, BC), lambda i, j: (i + ro, j)),
        ],
        out_shape=[
            jax.ShapeDtypeStruct((m_tok, SELW), jnp.float32),
            jax.ShapeDtypeStruct((m_tok, SELW), jnp.int32),
            jax.ShapeDtypeStruct((N_TOK, D_SAE), jnp.float32),
        ],
        scratch_shapes=[pltpu.VMEM((BR, SELW), jnp.float32)],
        input_output_aliases=aliases,
        compiler_params=pltpu.CompilerParams(
            dimension_semantics=("parallel", "arbitrary"),
        ),
    )(*args)


def kernel(x, W_enc, b_enc, dec_bias):
    xc = x - dec_bias[None, :]
    b2 = b_enc.reshape(1, -1)
    nchunk = 4
    half = N_TOK // nchunk
    tops = []
    enc = None
    posts = [_encode_post(xc[h * half:(h + 1) * half], W_enc, b2)
             for h in range(nchunk)]
    sels = [_topk_select(lax.bitcast_convert_type(p, jnp.int32))
            for p in posts]
    for h in range(nchunk):
        sv, si = sels[h]
        tv, ti, enc = _rank_and_mask(
            lax.bitcast_convert_type(sv, jnp.float32), si, posts[h],
            h * half, enc)
        tops.append((tv, ti))
    top_v = jnp.concatenate([t[0] for t in tops], axis=0)
    top_i = jnp.concatenate([t[1] for t in tops], axis=0)
    return enc, top_v[:, :TOPK], top_i[:, :TOPK]
